# baseline (speedup 1.0000x reference)
import jax
import jax.numpy as jnp
from jax.experimental import pallas as pl
from jax.experimental.pallas import tpu as pltpu

CHUNKS = (256, 256, 512, 512, 512)


def _make_body(batch, seq_len):
    starts = []
    off = 0
    for sz in CHUNKS:
        starts.append(off)
        off += sz
    assert off == seq_len

    def _body(w_hbm, o_hbm, buf, sem_in, sem_out):
        in_cps = []
        for i, (st, sz) in enumerate(zip(starts, CHUNKS)):
            cp = pltpu.make_async_copy(
                w_hbm.at[pl.ds(st, sz)], buf.at[pl.ds(st, sz)], sem_in.at[i]
            )
            cp.start()
            in_cps.append(cp)
        out_cps = []
        for i, (st, sz) in enumerate(zip(starts, CHUNKS)):
            in_cps[i].wait()
            for b in range(batch):
                cp = pltpu.make_async_copy(
                    buf.at[pl.ds(st, sz)],
                    o_hbm.at[b, pl.ds(st, sz)],
                    sem_out.at[i],
                )
                cp.start()
                out_cps.append(cp)
        for cp in out_cps:
            cp.wait()

    return _body


def kernel(tokens, W_pos):
    batch, seq_len = tokens.shape
    d_model = W_pos.shape[1]
    n = len(CHUNKS)
    return pl.pallas_call(
        _make_body(batch, seq_len),
        in_specs=[pl.BlockSpec(memory_space=pl.ANY)],
        out_specs=pl.BlockSpec(memory_space=pl.ANY),
        out_shape=jax.ShapeDtypeStruct((batch, seq_len, d_model), W_pos.dtype),
        scratch_shapes=[
            pltpu.VMEM((seq_len, d_model), jnp.float32),
            pltpu.SemaphoreType.DMA((n,)),
            pltpu.SemaphoreType.DMA((n,)),
        ],
    )(W_pos)


# speedup vs baseline: 1.0019x; 1.0019x over previous
import jax
import jax.numpy as jnp
from jax.experimental import pallas as pl
from jax.experimental.pallas import tpu as pltpu

CHUNKS = (128, 256, 640, 1024)


def _make_body(batch, seq_len):
    starts = []
    off = 0
    for sz in CHUNKS:
        starts.append(off)
        off += sz
    assert off == seq_len

    def _body(w_hbm, o_hbm, buf, sem_in, sem_out):
        in_cps = []
        for i, (st, sz) in enumerate(zip(starts, CHUNKS)):
            cp = pltpu.make_async_copy(
                w_hbm.at[pl.ds(st, sz)], buf.at[pl.ds(st, sz)], sem_in.at[i]
            )
            cp.start()
            in_cps.append(cp)
        out_cps = []
        for i, (st, sz) in enumerate(zip(starts, CHUNKS)):
            in_cps[i].wait()
            for b in range(batch):
                cp = pltpu.make_async_copy(
                    buf.at[pl.ds(st, sz)],
                    o_hbm.at[b, pl.ds(st, sz)],
                    sem_out.at[i],
                )
                cp.start()
                out_cps.append(cp)
        for cp in out_cps:
            cp.wait()

    return _body


def kernel(tokens, W_pos):
    batch, seq_len = tokens.shape
    d_model = W_pos.shape[1]
    n = len(CHUNKS)
    return pl.pallas_call(
        _make_body(batch, seq_len),
        in_specs=[pl.BlockSpec(memory_space=pl.ANY)],
        out_specs=pl.BlockSpec(memory_space=pl.ANY),
        out_shape=jax.ShapeDtypeStruct((batch, seq_len, d_model), W_pos.dtype),
        scratch_shapes=[
            pltpu.VMEM((seq_len, d_model), jnp.float32),
            pltpu.SemaphoreType.DMA((n,)),
            pltpu.SemaphoreType.DMA((n,)),
        ],
    )(W_pos)


# speedup vs baseline: 1.0092x; 1.0073x over previous
import jax
import jax.numpy as jnp
from jax.experimental import pallas as pl
from jax.experimental.pallas import tpu as pltpu

CHUNKS = (256, 256, 512, 1024)


def _make_body(batch, seq_len):
    starts = []
    off = 0
    for sz in CHUNKS:
        starts.append(off)
        off += sz
    assert off == seq_len

    def _body(w_hbm, o_hbm, buf, sem_in, sem_out):
        in_cps = []
        for i, (st, sz) in enumerate(zip(starts, CHUNKS)):
            cp = pltpu.make_async_copy(
                w_hbm.at[pl.ds(st, sz)], buf.at[pl.ds(st, sz)], sem_in.at[i]
            )
            cp.start()
            in_cps.append(cp)
        out_cps = []
        for i, (st, sz) in enumerate(zip(starts, CHUNKS)):
            in_cps[i].wait()
            for bb in range(batch):
                cp = pltpu.make_async_copy(
                    buf.at[pl.ds(st, sz)],
                    o_hbm.at[(i + bb) % batch, pl.ds(st, sz)],
                    sem_out.at[i],
                )
                cp.start()
                out_cps.append(cp)
        for cp in out_cps:
            cp.wait()

    return _body


def kernel(tokens, W_pos):
    batch, seq_len = tokens.shape
    d_model = W_pos.shape[1]
    n = len(CHUNKS)
    return pl.pallas_call(
        _make_body(batch, seq_len),
        in_specs=[pl.BlockSpec(memory_space=pl.ANY)],
        out_specs=pl.BlockSpec(memory_space=pl.ANY),
        out_shape=jax.ShapeDtypeStruct((batch, seq_len, d_model), W_pos.dtype),
        scratch_shapes=[
            pltpu.VMEM((seq_len, d_model), jnp.float32),
            pltpu.SemaphoreType.DMA((n,)),
            pltpu.SemaphoreType.DMA((n,)),
        ],
    )(W_pos)


# speedup vs baseline: 1.0154x; 1.0061x over previous
import jax
import jax.numpy as jnp
from jax.experimental import pallas as pl
from jax.experimental.pallas import tpu as pltpu

CHUNKS = (256, 256, 512, 1024)


def _make_body(batch, seq_len):
    chunks = CHUNKS if sum(CHUNKS) == seq_len else (seq_len // 4,) * 4
    starts = []
    off = 0
    for sz in chunks:
        starts.append(off)
        off += sz
    assert off == seq_len

    def _body(w_hbm, o_hbm, buf, sem_in, sem_out):
        in_cps = []
        for i, (st, sz) in enumerate(zip(starts, chunks)):
            cp = pltpu.make_async_copy(
                w_hbm.at[pl.ds(st, sz)], buf.at[pl.ds(st, sz)], sem_in.at[i]
            )
            cp.start()
            in_cps.append(cp)
        out_cps = []
        for i, (st, sz) in enumerate(zip(starts, chunks)):
            in_cps[i].wait()
            for bb in range(batch):
                cp = pltpu.make_async_copy(
                    buf.at[pl.ds(st, sz)],
                    o_hbm.at[(i + bb) % batch, pl.ds(st, sz)],
                    sem_out.at[i],
                )
                cp.start()
                out_cps.append(cp)
        for cp in out_cps:
            cp.wait()

    return _body


def kernel(tokens, W_pos):
    batch, seq_len = tokens.shape
    d_model = W_pos.shape[1]
    n = max(len(CHUNKS), 4)
    return pl.pallas_call(
        _make_body(batch, seq_len),
        in_specs=[pl.BlockSpec(memory_space=pl.ANY)],
        out_specs=pl.BlockSpec(memory_space=pl.ANY),
        out_shape=jax.ShapeDtypeStruct((batch, seq_len, d_model), W_pos.dtype),
        scratch_shapes=[
            pltpu.VMEM((seq_len, d_model), jnp.float32),
            pltpu.SemaphoreType.DMA((n,)),
            pltpu.SemaphoreType.DMA((n,)),
        ],
    )(W_pos)
